# baseline (speedup 1.0000x reference)
import functools

import jax
import jax.numpy as jnp
from jax import lax
from jax.experimental import pallas as pl
from jax.experimental.pallas import tpu as pltpu
from jax.experimental.pallas import tpu_sc as plsc


def _sc_gather(t_sta, t_net, t_cha, t_sen, i_sta, i_net, i_cha, i_sen):
    B = i_sta.shape[0]
    info = plsc.get_sparse_core_info()
    NC, NS = info.num_cores, info.num_subcores
    NW = NC * NS
    bw = B // NW
    dt = t_sta.dtype
    mesh = plsc.VectorSubcoreMesh(core_axis_name="c", subcore_axis_name="s")

    @functools.partial(
        pl.kernel,
        mesh=mesh,
        compiler_params=pltpu.CompilerParams(use_tc_tiling_on_sc=False),
        out_type=[
            jax.ShapeDtypeStruct((B, 32), dt),
            jax.ShapeDtypeStruct((B, 32), dt),
            jax.ShapeDtypeStruct((B, 16), dt),
            jax.ShapeDtypeStruct((B, 16), dt),
        ],
        scratch_types=[
            pltpu.VMEM((bw,), jnp.int32),
            pltpu.VMEM((bw,), jnp.int32),
            pltpu.VMEM((bw,), jnp.int32),
            pltpu.VMEM((bw,), jnp.int32),
            pltpu.VMEM((bw, 32), dt),
            pltpu.VMEM((bw, 32), dt),
            pltpu.VMEM((bw, 16), dt),
            pltpu.VMEM((bw, 16), dt),
            pltpu.SemaphoreType.DMA,
            pltpu.SemaphoreType.DMA,
            pltpu.SemaphoreType.DMA,
            pltpu.SemaphoreType.DMA,
        ],
    )
    def gather_k(ts_h, tn_h, tc_h, te_h, is_h, in_h, ic_h, ie_h,
                 os_h, on_h, oc_h, oe_h,
                 iv_s, iv_n, iv_c, iv_e, rv_s, rv_n, rv_c, rv_e,
                 sem_s, sem_n, sem_c, sem_e):
        wid = lax.axis_index("s") * NC + lax.axis_index("c")
        base = wid * bw
        pltpu.sync_copy(is_h.at[pl.ds(base, bw)], iv_s)
        pltpu.sync_copy(in_h.at[pl.ds(base, bw)], iv_n)
        pltpu.sync_copy(ic_h.at[pl.ds(base, bw)], iv_c)
        pltpu.sync_copy(ie_h.at[pl.ds(base, bw)], iv_e)
        cp_s = pltpu.async_copy(ts_h.at[iv_s], rv_s, sem_s)
        cp_n = pltpu.async_copy(tn_h.at[iv_n], rv_n, sem_n)
        cp_c = pltpu.async_copy(tc_h.at[iv_c], rv_c, sem_c)
        cp_e = pltpu.async_copy(te_h.at[iv_e], rv_e, sem_e)
        cp_s.wait()
        pltpu.sync_copy(rv_s, os_h.at[pl.ds(base, bw)])
        cp_n.wait()
        pltpu.sync_copy(rv_n, on_h.at[pl.ds(base, bw)])
        cp_c.wait()
        pltpu.sync_copy(rv_c, oc_h.at[pl.ds(base, bw)])
        cp_e.wait()
        pltpu.sync_copy(rv_e, oe_h.at[pl.ds(base, bw)])

    return gather_k(t_sta, t_net, t_cha, t_sen, i_sta, i_net, i_cha, i_sen)


def _dense_body(es_r, en_r, ec_r, ee_r, cont_r,
                w1_r, b1_r, w2_r, b2_r, wp1_r, bp1_r, wp2_r, bp2_r,
                out_r):
    f32 = jnp.float32
    es = es_r[...].astype(f32)
    en = en_r[...].astype(f32)
    ec = ec_r[...].astype(f32)
    ee = ee_r[...].astype(f32)
    h = jnp.dot(cont_r[...], w1_r[...], preferred_element_type=f32) + b1_r[...]
    h = jnp.maximum(h, 0.0)
    h = jnp.dot(h, w2_r[...], preferred_element_type=f32) + b2_r[...]
    h = jnp.maximum(h, 0.0)
    p = (jnp.dot(es, wp1_r[0:32, :], preferred_element_type=f32)
         + jnp.dot(en, wp1_r[32:64, :], preferred_element_type=f32)
         + jnp.dot(ec, wp1_r[64:80, :], preferred_element_type=f32)
         + jnp.dot(ee, wp1_r[80:96, :], preferred_element_type=f32)
         + jnp.dot(h, wp1_r[96:224, :], preferred_element_type=f32)
         + bp1_r[...])
    p = jnp.maximum(p, 0.0)
    out_r[...] = (jnp.dot(p, wp2_r[...], preferred_element_type=f32)
                  + bp2_r[...])


def _tc_dense(es, en, ec, ee, cont, W1, b1, W2, b2, Wp1, bp1, Wp2, bp2):
    B = es.shape[0]
    BM = 2048
    grid = (B // BM,)

    def row_spec(n):
        return pl.BlockSpec((BM, n), lambda i: (i, 0))

    def full_spec(m, n):
        return pl.BlockSpec((m, n), lambda i: (0, 0))

    return pl.pallas_call(
        _dense_body,
        grid=grid,
        in_specs=[
            row_spec(32), row_spec(32), row_spec(16), row_spec(16),
            row_spec(3),
            full_spec(3, 128), full_spec(1, 128),
            full_spec(128, 128), full_spec(1, 128),
            full_spec(224, 128), full_spec(1, 128),
            full_spec(128, 128), full_spec(1, 128),
        ],
        out_specs=row_spec(128),
        out_shape=jax.ShapeDtypeStruct((B, 128), jnp.float32),
    )(es, en, ec, ee, cont, W1, b1, W2, b2, Wp1, bp1, Wp2, bp2)


def kernel(station_id, network_id, channel_code, sensor_type,
           latitude, longitude, elevation,
           T_station, T_network, T_channel, T_sensor,
           W1, b1, W2, b2, Wp1, bp1, Wp2, bp2):
    bf = jnp.bfloat16
    es, en, ec, ee = _sc_gather(
        T_station.astype(bf), T_network.astype(bf),
        T_channel.astype(bf), T_sensor.astype(bf),
        station_id.astype(jnp.int32), network_id.astype(jnp.int32),
        channel_code.astype(jnp.int32), sensor_type.astype(jnp.int32))
    cont = jnp.stack([latitude, longitude, elevation], axis=-1)
    return _tc_dense(es, en, ec, ee, cont,
                     W1, b1.reshape(1, -1), W2, b2.reshape(1, -1),
                     Wp1, bp1.reshape(1, -1), Wp2, bp2.reshape(1, -1))


# speedup vs baseline: 1.0994x; 1.0994x over previous
import functools

import jax
import jax.numpy as jnp
from jax import lax
from jax.experimental import pallas as pl
from jax.experimental.pallas import tpu as pltpu
from jax.experimental.pallas import tpu_sc as plsc


def _sc_gather(t_s, t_n, t_c, t_e, p_s, p_n, p_c, p_e):
    B = p_s.shape[0]
    info = plsc.get_sparse_core_info()
    NC, NS = info.num_cores, info.num_subcores
    NW = NC * NS
    bw = B // NW
    CH = 256
    NCH = bw // CH
    mesh = plsc.VectorSubcoreMesh(core_axis_name="c", subcore_axis_name="s")

    @functools.partial(
        pl.kernel,
        mesh=mesh,
        compiler_params=pltpu.CompilerParams(use_tc_tiling_on_sc=True),
        out_type=[jax.ShapeDtypeStruct((B, 128), jnp.float32)
                  for _ in range(4)],
        scratch_types=[
            pltpu.VMEM((bw,), jnp.int32),
            pltpu.VMEM((bw,), jnp.int32),
            pltpu.VMEM((bw,), jnp.int32),
            pltpu.VMEM((bw,), jnp.int32),
            pltpu.VMEM((CH, 128), jnp.float32),
            pltpu.VMEM((CH, 128), jnp.float32),
            pltpu.SemaphoreType.DMA,
            pltpu.SemaphoreType.DMA,
        ],
    )
    def gather_k(ts_h, tn_h, tc_h, te_h, ps_h, pn_h, pc_h, pe_h,
                 gs_h, gn_h, gc_h, ge_h,
                 iv_s, iv_n, iv_c, iv_e, bufa, bufb, sema, semb):
        wid = lax.axis_index("s") * NC + lax.axis_index("c")
        base = wid * bw
        pltpu.sync_copy(ps_h.at[pl.ds(base, bw)], iv_s)
        pltpu.sync_copy(pn_h.at[pl.ds(base, bw)], iv_n)
        pltpu.sync_copy(pc_h.at[pl.ds(base, bw)], iv_c)
        pltpu.sync_copy(pe_h.at[pl.ds(base, bw)], iv_e)
        steps = []
        for tbl, iv, out in ((ts_h, iv_s, gs_h), (tn_h, iv_n, gn_h),
                             (tc_h, iv_c, gc_h), (te_h, iv_e, ge_h)):
            for c in range(NCH):
                steps.append((tbl, iv, out, c))
        bufs = (bufa, bufb)
        sems = (sema, semb)
        pend = [None, None]
        for k, (tbl, iv, out, c) in enumerate(steps):
            slot = k % 2
            if pend[slot] is not None:
                cp, pout, pc_ = pend[slot]
                cp.wait()
                pltpu.sync_copy(bufs[slot],
                                pout.at[pl.ds(base + pc_ * CH, CH)])
            cp = pltpu.async_copy(tbl.at[iv.at[pl.ds(c * CH, CH)]],
                                  bufs[slot], sems[slot])
            pend[slot] = (cp, out, c)
        for slot in (0, 1):
            if pend[slot] is not None:
                cp, pout, pc_ = pend[slot]
                cp.wait()
                pltpu.sync_copy(bufs[slot],
                                pout.at[pl.ds(base + pc_ * CH, CH)])

    return gather_k(t_s, t_n, t_c, t_e, p_s, p_n, p_c, p_e)


def _dense_body(gs_r, gn_r, gc_r, ge_r, qs_r, qn_r, qc_r, qe_r, cont_r,
                w1_r, b1_r, w2_r, b2_r,
                wps_r, wpn_r, wpc_r, wpe_r, wph_r, bp1_r, wp2_r, bp2_r,
                out_r):
    f32 = jnp.float32
    shape = gs_r.shape
    lane = jax.lax.broadcasted_iota(jnp.int32, shape, 1)
    g32 = lane // 32
    g16 = lane // 16
    ms = (g32 == qs_r[...]).astype(f32)
    mn = (g32 == qn_r[...]).astype(f32)
    mc = (g16 == qc_r[...]).astype(f32)
    me = (g16 == qe_r[...]).astype(f32)
    h = jnp.dot(cont_r[...], w1_r[...], preferred_element_type=f32) + b1_r[...]
    h = jnp.maximum(h, 0.0)
    h = jnp.dot(h, w2_r[...], preferred_element_type=f32) + b2_r[...]
    h = jnp.maximum(h, 0.0)
    p = (jnp.dot(gs_r[...] * ms, wps_r[...], preferred_element_type=f32)
         + jnp.dot(gn_r[...] * mn, wpn_r[...], preferred_element_type=f32)
         + jnp.dot(gc_r[...] * mc, wpc_r[...], preferred_element_type=f32)
         + jnp.dot(ge_r[...] * me, wpe_r[...], preferred_element_type=f32)
         + jnp.dot(h, wph_r[...], preferred_element_type=f32)
         + bp1_r[...])
    p = jnp.maximum(p, 0.0)
    out_r[...] = (jnp.dot(p, wp2_r[...], preferred_element_type=f32)
                  + bp2_r[...])


def _tc_dense(gs, gn, gc, ge, qs, qn, qc, qe, cont,
              W1, b1, W2, b2, Wps, Wpn, Wpc, Wpe, Wph, bp1, Wp2, bp2):
    B = gs.shape[0]
    BM = 2048
    grid = (B // BM,)

    def row_spec(n):
        return pl.BlockSpec((BM, n), lambda i: (i, 0))

    def full_spec(m, n):
        return pl.BlockSpec((m, n), lambda i: (0, 0))

    return pl.pallas_call(
        _dense_body,
        grid=grid,
        in_specs=[
            row_spec(128), row_spec(128), row_spec(128), row_spec(128),
            row_spec(1), row_spec(1), row_spec(1), row_spec(1),
            row_spec(3),
            full_spec(3, 128), full_spec(1, 128),
            full_spec(128, 128), full_spec(1, 128),
            full_spec(128, 128), full_spec(128, 128),
            full_spec(128, 128), full_spec(128, 128),
            full_spec(128, 128), full_spec(1, 128),
            full_spec(128, 128), full_spec(1, 128),
        ],
        out_specs=row_spec(128),
        out_shape=jax.ShapeDtypeStruct((B, 128), jnp.float32),
    )(gs, gn, gc, ge, qs, qn, qc, qe, cont,
      W1, b1, W2, b2, Wps, Wpn, Wpc, Wpe, Wph, bp1, Wp2, bp2)


def kernel(station_id, network_id, channel_code, sensor_type,
           latitude, longitude, elevation,
           T_station, T_network, T_channel, T_sensor,
           W1, b1, W2, b2, Wp1, bp1, Wp2, bp2):
    i32 = jnp.int32
    i_s = station_id.astype(i32)
    i_n = network_id.astype(i32)
    i_c = channel_code.astype(i32)
    i_e = sensor_type.astype(i32)
    gs, gn, gc, ge = _sc_gather(
        T_station.reshape(-1, 128), T_network.reshape(-1, 128),
        T_channel.reshape(-1, 128), T_sensor.reshape(-1, 128),
        i_s >> 2, i_n >> 2, i_c >> 3, i_e >> 3)
    cont = jnp.stack([latitude, longitude, elevation], axis=-1)
    Wps = jnp.concatenate([Wp1[0:32]] * 4, axis=0)
    Wpn = jnp.concatenate([Wp1[32:64]] * 4, axis=0)
    Wpc = jnp.concatenate([Wp1[64:80]] * 8, axis=0)
    Wpe = jnp.concatenate([Wp1[80:96]] * 8, axis=0)
    return _tc_dense(gs, gn, gc, ge,
                     (i_s & 3).reshape(-1, 1), (i_n & 3).reshape(-1, 1),
                     (i_c & 7).reshape(-1, 1), (i_e & 7).reshape(-1, 1),
                     cont, W1, b1.reshape(1, -1), W2, b2.reshape(1, -1),
                     Wps, Wpn, Wpc, Wpe, Wp1[96:224], bp1.reshape(1, -1),
                     Wp2, bp2.reshape(1, -1))


# speedup vs baseline: 1.7866x; 1.6250x over previous
import functools

import jax
import jax.numpy as jnp
from jax import lax
from jax.experimental import pallas as pl
from jax.experimental.pallas import tpu as pltpu
from jax.experimental.pallas import tpu_sc as plsc
from jax.experimental.layout import Format, Layout, with_layout_constraint


def _to_sc_layout(x):
    return with_layout_constraint(
        x, Layout(major_to_minor=(0, 1), tiling=((8,),)))


def _sc_gather(t_sta, t_net, t_cha, t_sen, i_sta, i_net, i_cha, i_sen):
    B = i_sta.shape[0]
    info = plsc.get_sparse_core_info()
    NC, NS = info.num_cores, info.num_subcores
    NW = NC * NS
    bw = B // NW
    mesh = plsc.VectorSubcoreMesh(core_axis_name="c", subcore_axis_name="s")

    @functools.partial(
        pl.kernel,
        mesh=mesh,
        compiler_params=pltpu.CompilerParams(use_tc_tiling_on_sc=False),
        out_type=[
            jax.ShapeDtypeStruct((B, 32), jnp.float32),
            jax.ShapeDtypeStruct((B, 32), jnp.float32),
            jax.ShapeDtypeStruct((B, 16), jnp.float32),
            jax.ShapeDtypeStruct((B, 16), jnp.float32),
        ],
        scratch_types=[
            pltpu.VMEM((bw,), jnp.int32),
            pltpu.VMEM((bw,), jnp.int32),
            pltpu.VMEM((bw,), jnp.int32),
            pltpu.VMEM((bw,), jnp.int32),
            pltpu.VMEM((bw, 32), jnp.float32),
            pltpu.VMEM((bw, 32), jnp.float32),
            pltpu.VMEM((bw, 16), jnp.float32),
            pltpu.VMEM((bw, 16), jnp.float32),
            pltpu.SemaphoreType.DMA,
            pltpu.SemaphoreType.DMA,
            pltpu.SemaphoreType.DMA,
            pltpu.SemaphoreType.DMA,
        ],
    )
    def gather_k(ts_h, tn_h, tc_h, te_h, is_h, in_h, ic_h, ie_h,
                 os_h, on_h, oc_h, oe_h,
                 iv_s, iv_n, iv_c, iv_e, rv_s, rv_n, rv_c, rv_e,
                 sem_s, sem_n, sem_c, sem_e):
        wid = lax.axis_index("s") * NC + lax.axis_index("c")
        base = wid * bw
        pltpu.sync_copy(is_h.at[pl.ds(base, bw)], iv_s)
        pltpu.sync_copy(in_h.at[pl.ds(base, bw)], iv_n)
        pltpu.sync_copy(ic_h.at[pl.ds(base, bw)], iv_c)
        pltpu.sync_copy(ie_h.at[pl.ds(base, bw)], iv_e)
        cp_s = pltpu.async_copy(ts_h.at[iv_s], rv_s, sem_s)
        cp_n = pltpu.async_copy(tn_h.at[iv_n], rv_n, sem_n)
        cp_c = pltpu.async_copy(tc_h.at[iv_c], rv_c, sem_c)
        cp_e = pltpu.async_copy(te_h.at[iv_e], rv_e, sem_e)
        cp_s.wait()
        pltpu.sync_copy(rv_s, os_h.at[pl.ds(base, bw)])
        cp_n.wait()
        pltpu.sync_copy(rv_n, on_h.at[pl.ds(base, bw)])
        cp_c.wait()
        pltpu.sync_copy(rv_c, oc_h.at[pl.ds(base, bw)])
        cp_e.wait()
        pltpu.sync_copy(rv_e, oe_h.at[pl.ds(base, bw)])

    return gather_k(t_sta, t_net, t_cha, t_sen, i_sta, i_net, i_cha, i_sen)


def _dense_body(es_r, en_r, ec_r, ee_r, cont_r,
                w1_r, b1_r, w2_r, b2_r, wp1_r, bp1_r, wp2_r, bp2_r,
                out_r):
    f32 = jnp.float32
    h = jnp.dot(cont_r[...], w1_r[...], preferred_element_type=f32) + b1_r[...]
    h = jnp.maximum(h, 0.0)
    h = jnp.dot(h, w2_r[...], preferred_element_type=f32) + b2_r[...]
    h = jnp.maximum(h, 0.0)
    p = (jnp.dot(es_r[...], wp1_r[0:32, :], preferred_element_type=f32)
         + jnp.dot(en_r[...], wp1_r[32:64, :], preferred_element_type=f32)
         + jnp.dot(ec_r[...], wp1_r[64:80, :], preferred_element_type=f32)
         + jnp.dot(ee_r[...], wp1_r[80:96, :], preferred_element_type=f32)
         + jnp.dot(h, wp1_r[96:224, :], preferred_element_type=f32)
         + bp1_r[...])
    p = jnp.maximum(p, 0.0)
    out_r[...] = (jnp.dot(p, wp2_r[...], preferred_element_type=f32)
                  + bp2_r[...])


def _tc_dense(es, en, ec, ee, cont, W1, b1, W2, b2, Wp1, bp1, Wp2, bp2):
    B = es.shape[0]
    BM = 2048
    grid = (B // BM,)

    def row_spec(n):
        return pl.BlockSpec((BM, n), lambda i: (i, 0))

    def full_spec(m, n):
        return pl.BlockSpec((m, n), lambda i: (0, 0))

    return pl.pallas_call(
        _dense_body,
        grid=grid,
        in_specs=[
            row_spec(32), row_spec(32), row_spec(16), row_spec(16),
            row_spec(3),
            full_spec(3, 128), full_spec(1, 128),
            full_spec(128, 128), full_spec(1, 128),
            full_spec(224, 128), full_spec(1, 128),
            full_spec(128, 128), full_spec(1, 128),
        ],
        out_specs=row_spec(128),
        out_shape=jax.ShapeDtypeStruct((B, 128), jnp.float32),
    )(es, en, ec, ee, cont, W1, b1, W2, b2, Wp1, bp1, Wp2, bp2)


def kernel(station_id, network_id, channel_code, sensor_type,
           latitude, longitude, elevation,
           T_station, T_network, T_channel, T_sensor,
           W1, b1, W2, b2, Wp1, bp1, Wp2, bp2):
    es, en, ec, ee = _sc_gather(
        _to_sc_layout(T_station), _to_sc_layout(T_network),
        _to_sc_layout(T_channel), _to_sc_layout(T_sensor),
        station_id.astype(jnp.int32), network_id.astype(jnp.int32),
        channel_code.astype(jnp.int32), sensor_type.astype(jnp.int32))
    cont = jnp.stack([latitude, longitude, elevation], axis=-1)
    return _tc_dense(es, en, ec, ee, cont,
                     W1, b1.reshape(1, -1), W2, b2.reshape(1, -1),
                     Wp1, bp1.reshape(1, -1), Wp2, bp2.reshape(1, -1))
